# baseline (speedup 1.0000x reference)
import functools

import jax
import jax.numpy as jnp
from jax import lax
from jax.experimental import pallas as pl
from jax.experimental.pallas import tpu as pltpu
from jax.experimental.pallas import tpu_sc as plsc

EMB = 64
LANES = 16
CHUNK = 128


@functools.cache
def _sc_embed_dot(b_per_w, batch):
    n_chunks = b_per_w // CHUNK
    n_groups = b_per_w // LANES
    mesh = plsc.VectorSubcoreMesh(core_axis_name="c", subcore_axis_name="s")

    @functools.partial(
        pl.kernel,
        mesh=mesh,
        out_type=jax.ShapeDtypeStruct((batch,), jnp.float32),
        compiler_params=pltpu.CompilerParams(needs_layout_passes=False,
                                             use_tc_tiling_on_sc=False),
        scratch_types=[
            pltpu.VMEM((n_chunks, CHUNK), jnp.int32),
            pltpu.VMEM((n_chunks, CHUNK), jnp.int32),
            pltpu.VMEM((b_per_w, EMB), jnp.float32),
            pltpu.VMEM((b_per_w, EMB), jnp.float32),
            pltpu.VMEM((LANES,), jnp.float32),
            pltpu.VMEM((EMB, LANES), jnp.float32),
            pltpu.VMEM((b_per_w,), jnp.float32),
            pltpu.SemaphoreType.DMA,
        ],
    )
    def sc_fn(user_hbm, item_hbm, ut_hbm, it_hbm, wb_hbm, b_hbm, out_hbm,
              uidx_v, iidx_v, urows_v, irows_v, b_v, wb_v, out_v, sem):
        num_cores = 2
        wid = lax.axis_index("s") * num_cores + lax.axis_index("c")
        base = wid * b_per_w

        for j in range(n_chunks):
            off = base + j * CHUNK
            pltpu.sync_copy(user_hbm.at[pl.ds(off, CHUNK)], uidx_v.at[j])
            pltpu.sync_copy(item_hbm.at[pl.ds(off, CHUNK)], iidx_v.at[j])

        descs = []
        for j in range(n_chunks):
            dst = pl.ds(j * CHUNK, CHUNK)
            descs.append(
                pltpu.async_copy(ut_hbm.at[uidx_v.at[j]], urows_v.at[dst], sem))
            descs.append(
                pltpu.async_copy(it_hbm.at[iidx_v.at[j]], irows_v.at[dst], sem))

        pltpu.sync_copy(wb_hbm, wb_v)
        pltpu.sync_copy(b_hbm, b_v)

        for d in descs:
            d.wait()

        bias = b_v[...]
        lane_iota = lax.iota(jnp.int32, LANES)

        def group_body(g, carry):
            row_idx = g * LANES + lane_iota
            acc = bias
            for k in range(EMB):
                ck = jnp.full((LANES,), k, dtype=jnp.int32)
                gu = plsc.load_gather(urows_v, [row_idx, ck])
                gv = plsc.load_gather(irows_v, [row_idx, ck])
                acc = acc + gu * gv * wb_v[k]
            out_v[pl.ds(pl.multiple_of(g * LANES, LANES), LANES)] = acc
            return carry

        lax.fori_loop(0, n_groups, group_body, 0)

        pltpu.sync_copy(out_v, out_hbm.at[pl.ds(base, b_per_w)])

    return sc_fn


def kernel(user, item, user_table, item_table, W, b):
    batch = user.shape[0]
    num_workers = 32
    b_per_w = batch // num_workers
    wb = jnp.broadcast_to(W.reshape(EMB, 1), (EMB, LANES))
    b16 = jnp.broadcast_to(b, (LANES,))
    fn = _sc_embed_dot(b_per_w, batch)
    out = fn(user, item, user_table, item_table, wb, b16)
    return out.reshape(batch, 1)
